# Initial kernel scaffold; baseline (speedup 1.0000x reference)
import jax
import jax.numpy as jnp
from jax.experimental import pallas as pl


def kernel(x, edge_index, W_l, b_l, W_r, gamma, beta):
    raise NotImplementedError("write your pallas kernel here")


# speedup vs baseline: 5.9993x; 5.9993x over previous
import functools

import jax
import jax.numpy as jnp
from jax import lax
from jax.experimental import pallas as pl
from jax.experimental.pallas import tpu as pltpu
from jax.experimental.pallas import tpu_sc as plsc

N = 10000
E = 320000
D = 128

NC = 2
NS = 16
DH = D // NC
C = 80
CH = E // (NS * C)
NPAD = 10240
ROWS_PER_TILE = NPAD // NS


def _sc_aggregate_body(xh_hbm, src_hbm, dst_hbm, agg_out, deg_out,
                       src_v, dst_v, rows_v, ones_v, zeros_v,
                       agg_sh, deg_sh, sem):
    cid = lax.axis_index("c")
    sid = lax.axis_index("s")

    pltpu.sync_copy(src_hbm.at[sid], src_v)
    pltpu.sync_copy(dst_hbm.at[sid], dst_v)

    zeros16 = jnp.zeros((16,), jnp.float32)
    ones16 = jnp.ones((16,), jnp.float32)

    def fill_row(r, _):
        def fill_col(k, _):
            rows_v[r, pl.ds(k * 16, 16)] = zeros16
            return 0
        lax.fori_loop(0, DH // 16, fill_col, 0)
        ones_v[r, pl.ds(0, 16)] = ones16
        zeros_v[r, pl.ds(0, 16)] = zeros16
        return 0
    lax.fori_loop(0, C, fill_row, 0)

    for j in range(ROWS_PER_TILE // C):
        pltpu.sync_copy(rows_v, agg_sh.at[pl.ds(sid * ROWS_PER_TILE + j * C, C)])
        pltpu.sync_copy(zeros_v, deg_sh.at[pl.ds(sid * ROWS_PER_TILE + j * C, C)])
    plsc.subcore_barrier()

    def chunk(j, _):
        pltpu.async_copy(xh_hbm.at[cid].at[src_v.at[j]], rows_v, sem).wait()
        pltpu.sync_copy(rows_v, agg_sh.at[dst_v.at[j]], add=True)

        @pl.when((j % 2) == cid)
        def _deg():
            pltpu.sync_copy(ones_v, deg_sh.at[dst_v.at[j]], add=True)
        return 0
    lax.fori_loop(0, CH, chunk, 0)

    plsc.subcore_barrier()

    pltpu.sync_copy(agg_sh.at[pl.ds(sid * ROWS_PER_TILE, ROWS_PER_TILE)],
                    agg_out.at[cid, pl.ds(sid * ROWS_PER_TILE, ROWS_PER_TILE)])
    pltpu.sync_copy(deg_sh.at[pl.ds(sid * ROWS_PER_TILE, ROWS_PER_TILE)],
                    deg_out.at[cid, pl.ds(sid * ROWS_PER_TILE, ROWS_PER_TILE)])


_sc_aggregate = functools.partial(
    pl.kernel,
    out_type=(jax.ShapeDtypeStruct((NC, NPAD, DH), jnp.float32),
              jax.ShapeDtypeStruct((NC, NPAD, 16), jnp.float32)),
    mesh=plsc.VectorSubcoreMesh(core_axis_name="c", subcore_axis_name="s",
                                num_cores=NC, num_subcores=NS),
    scratch_types=[
        pltpu.VMEM((CH, C), jnp.int32),
        pltpu.VMEM((CH, C), jnp.int32),
        pltpu.VMEM((C, DH), jnp.float32),
        pltpu.VMEM((C, 16), jnp.float32),
        pltpu.VMEM((C, 16), jnp.float32),
        pltpu.VMEM_SHARED((NPAD, DH), jnp.float32),
        pltpu.VMEM_SHARED((NPAD, 16), jnp.float32),
        pltpu.SemaphoreType.DMA,
    ],
    compiler_params=pltpu.CompilerParams(use_tc_tiling_on_sc=False),
)(_sc_aggregate_body)


R = 2000
GSTEPS = N // R


def _tc_h_body(aggp_ref, degp_ref, x_ref, wl_ref, bl_ref, wr_ref,
               h_ref, stats_ref):
    i = pl.program_id(0)
    agg = jnp.concatenate([aggp_ref[0], aggp_ref[1]], axis=1)
    deg = (degp_ref[0] + degp_ref[1])[:, 0:1]
    x = x_ref[...]
    mean_agg = agg * (1.0 / jnp.maximum(deg, 1.0))
    dn = (((1,), (1,)), ((), ()))
    h = (lax.dot_general(mean_agg, wl_ref[...], dn,
                         precision=lax.Precision.HIGHEST,
                         preferred_element_type=jnp.float32)
         + bl_ref[...][None, :]
         + lax.dot_general(x, wr_ref[...], dn,
                           precision=lax.Precision.HIGHEST,
                           preferred_element_type=jnp.float32))
    h_ref[...] = h
    s1 = jnp.sum(h, axis=0, keepdims=True)
    s2 = jnp.sum(h * h, axis=0, keepdims=True)
    part = jnp.concatenate(
        [s1, s2, jnp.zeros((6, D), jnp.float32)], axis=0)

    @pl.when(i == 0)
    def _init():
        stats_ref[...] = part

    @pl.when(i > 0)
    def _acc():
        stats_ref[...] += part


_tc_h = pl.pallas_call(
    _tc_h_body,
    grid=(GSTEPS,),
    in_specs=[
        pl.BlockSpec((NC, R, DH), lambda i: (0, i, 0)),
        pl.BlockSpec((NC, R, 16), lambda i: (0, i, 0)),
        pl.BlockSpec((R, D), lambda i: (i, 0)),
        pl.BlockSpec((D, D), lambda i: (0, 0)),
        pl.BlockSpec((D,), lambda i: (0,)),
        pl.BlockSpec((D, D), lambda i: (0, 0)),
    ],
    out_specs=(
        pl.BlockSpec((R, D), lambda i: (i, 0)),
        pl.BlockSpec((8, D), lambda i: (0, 0)),
    ),
    out_shape=(jax.ShapeDtypeStruct((N, D), jnp.float32),
               jax.ShapeDtypeStruct((8, D), jnp.float32)),
)


def _tc_norm_body(h_ref, x_ref, stats_ref, g_ref, b_ref, o_ref):
    h = h_ref[...]
    mu = stats_ref[0:1, :] * (1.0 / N)
    var = stats_ref[1:2, :] * (1.0 / N) - mu * mu
    hn = (h - mu) * lax.rsqrt(var + 1e-5) * g_ref[...][None, :] + b_ref[...][None, :]
    o_ref[...] = jnp.maximum(hn, 0.0) + x_ref[...]


_tc_norm = pl.pallas_call(
    _tc_norm_body,
    grid=(GSTEPS,),
    in_specs=[
        pl.BlockSpec((R, D), lambda i: (i, 0)),
        pl.BlockSpec((R, D), lambda i: (i, 0)),
        pl.BlockSpec((8, D), lambda i: (0, 0)),
        pl.BlockSpec((D,), lambda i: (0,)),
        pl.BlockSpec((D,), lambda i: (0,)),
    ],
    out_specs=pl.BlockSpec((R, D), lambda i: (i, 0)),
    out_shape=jax.ShapeDtypeStruct((N, D), jnp.float32),
)


def kernel(x, edge_index, W_l, b_l, W_r, gamma, beta):
    src = edge_index[0].astype(jnp.int32).reshape(NS, CH, C)
    dst = edge_index[1].astype(jnp.int32).reshape(NS, CH, C)
    xh = x.reshape(N, NC, DH).transpose(1, 0, 2)
    aggp, degp = _sc_aggregate(xh, src, dst)
    h, stats = _tc_h(aggp, degp, x, W_l, b_l, W_r)
    return _tc_norm(h, x, stats, gamma, beta)


# speedup vs baseline: 9.0140x; 1.5025x over previous
import functools

import jax
import jax.numpy as jnp
from jax import lax
from jax.experimental import pallas as pl
from jax.experimental.pallas import tpu as pltpu
from jax.experimental.pallas import tpu_sc as plsc

N = 10000
E = 320000
D = 128

NC = 2
NS = 16
DH = D // NC
C = 80
CH = E // (NS * C)
NPAD = 10240
ROWS_PER_TILE = NPAD // NS


def _sc_aggregate_body(xh_hbm, src_hbm, dst_hbm, agg_out, deg_out,
                       src_v, dst_v, rows_v, rows2_v, ones_v, zeros_v,
                       agg_sh, deg_sh, sem, sem2):
    cid = lax.axis_index("c")
    sid = lax.axis_index("s")

    pltpu.sync_copy(src_hbm.at[sid], src_v)
    pltpu.sync_copy(dst_hbm.at[sid], dst_v)

    zeros16 = jnp.zeros((16,), jnp.float32)
    ones16 = jnp.ones((16,), jnp.float32)

    def fill_row(r, _):
        def fill_col(k, _):
            rows_v[r, pl.ds(k * 16, 16)] = zeros16
            return 0
        lax.fori_loop(0, DH // 16, fill_col, 0)
        ones_v[r, pl.ds(0, 16)] = ones16
        zeros_v[r, pl.ds(0, 16)] = zeros16
        return 0
    lax.fori_loop(0, C, fill_row, 0)

    for j in range(ROWS_PER_TILE // C):
        pltpu.sync_copy(rows_v, agg_sh.at[pl.ds(sid * ROWS_PER_TILE + j * C, C)])
        pltpu.sync_copy(zeros_v, deg_sh.at[pl.ds(sid * ROWS_PER_TILE + j * C, C)])
    plsc.subcore_barrier()

    bufs = (rows_v, rows2_v)
    sems = (sem, sem2)

    def _gather(j, b):
        pltpu.async_copy(xh_hbm.at[cid].at[src_v.at[j]], bufs[b], sems[b])

    def _wait(j, b):
        pltpu.make_async_copy(xh_hbm.at[cid].at[src_v.at[j]], bufs[b],
                              sems[b]).wait()

    _gather(0, 0)

    def pair(i, _):
        j0 = 2 * i
        _gather(j0 + 1, 1)
        _wait(j0, 0)
        pltpu.sync_copy(bufs[0], agg_sh.at[dst_v.at[j0]], add=True)

        @pl.when(cid == 0)
        def _deg0():
            pltpu.sync_copy(ones_v, deg_sh.at[dst_v.at[j0]], add=True)

        @pl.when(j0 + 2 < CH)
        def _next():
            _gather(j0 + 2, 0)
        _wait(j0 + 1, 1)
        pltpu.sync_copy(bufs[1], agg_sh.at[dst_v.at[j0 + 1]], add=True)

        @pl.when(cid == 1)
        def _deg1():
            pltpu.sync_copy(ones_v, deg_sh.at[dst_v.at[j0 + 1]], add=True)
        return 0
    lax.fori_loop(0, CH // 2, pair, 0)

    plsc.subcore_barrier()

    pltpu.sync_copy(agg_sh.at[pl.ds(sid * ROWS_PER_TILE, ROWS_PER_TILE)],
                    agg_out.at[cid, pl.ds(sid * ROWS_PER_TILE, ROWS_PER_TILE)])
    pltpu.sync_copy(deg_sh.at[pl.ds(sid * ROWS_PER_TILE, ROWS_PER_TILE)],
                    deg_out.at[cid, pl.ds(sid * ROWS_PER_TILE, ROWS_PER_TILE)])


_sc_aggregate = functools.partial(
    pl.kernel,
    out_type=(jax.ShapeDtypeStruct((NC, NPAD, DH), jnp.float32),
              jax.ShapeDtypeStruct((NC, NPAD, 16), jnp.float32)),
    mesh=plsc.VectorSubcoreMesh(core_axis_name="c", subcore_axis_name="s",
                                num_cores=NC, num_subcores=NS),
    scratch_types=[
        pltpu.VMEM((CH, C), jnp.int32),
        pltpu.VMEM((CH, C), jnp.int32),
        pltpu.VMEM((C, DH), jnp.float32),
        pltpu.VMEM((C, DH), jnp.float32),
        pltpu.VMEM((C, 16), jnp.float32),
        pltpu.VMEM((C, 16), jnp.float32),
        pltpu.VMEM_SHARED((NPAD, DH), jnp.float32),
        pltpu.VMEM_SHARED((NPAD, 16), jnp.float32),
        pltpu.SemaphoreType.DMA,
        pltpu.SemaphoreType.DMA,
    ],
    compiler_params=pltpu.CompilerParams(use_tc_tiling_on_sc=False),
)(_sc_aggregate_body)


R = 2000
GSTEPS = N // R


def _tc_h_body(aggp_ref, degp_ref, x_ref, wl_ref, bl_ref, wr_ref,
               h_ref, stats_ref):
    i = pl.program_id(0)
    agg = jnp.concatenate([aggp_ref[0], aggp_ref[1]], axis=1)
    deg = (degp_ref[0] + degp_ref[1])[:, 0:1]
    x = x_ref[...]
    mean_agg = agg * (1.0 / jnp.maximum(deg, 1.0))
    dn = (((1,), (1,)), ((), ()))
    h = (lax.dot_general(mean_agg, wl_ref[...], dn,
                         precision=lax.Precision.HIGHEST,
                         preferred_element_type=jnp.float32)
         + bl_ref[...][None, :]
         + lax.dot_general(x, wr_ref[...], dn,
                           precision=lax.Precision.HIGHEST,
                           preferred_element_type=jnp.float32))
    h_ref[...] = h
    s1 = jnp.sum(h, axis=0, keepdims=True)
    s2 = jnp.sum(h * h, axis=0, keepdims=True)
    part = jnp.concatenate(
        [s1, s2, jnp.zeros((6, D), jnp.float32)], axis=0)

    @pl.when(i == 0)
    def _init():
        stats_ref[...] = part

    @pl.when(i > 0)
    def _acc():
        stats_ref[...] += part


_tc_h = pl.pallas_call(
    _tc_h_body,
    grid=(GSTEPS,),
    in_specs=[
        pl.BlockSpec((NC, R, DH), lambda i: (0, i, 0)),
        pl.BlockSpec((NC, R, 16), lambda i: (0, i, 0)),
        pl.BlockSpec((R, D), lambda i: (i, 0)),
        pl.BlockSpec((D, D), lambda i: (0, 0)),
        pl.BlockSpec((D,), lambda i: (0,)),
        pl.BlockSpec((D, D), lambda i: (0, 0)),
    ],
    out_specs=(
        pl.BlockSpec((R, D), lambda i: (i, 0)),
        pl.BlockSpec((8, D), lambda i: (0, 0)),
    ),
    out_shape=(jax.ShapeDtypeStruct((N, D), jnp.float32),
               jax.ShapeDtypeStruct((8, D), jnp.float32)),
)


def _tc_norm_body(h_ref, x_ref, stats_ref, g_ref, b_ref, o_ref):
    h = h_ref[...]
    mu = stats_ref[0:1, :] * (1.0 / N)
    var = stats_ref[1:2, :] * (1.0 / N) - mu * mu
    hn = (h - mu) * lax.rsqrt(var + 1e-5) * g_ref[...][None, :] + b_ref[...][None, :]
    o_ref[...] = jnp.maximum(hn, 0.0) + x_ref[...]


_tc_norm = pl.pallas_call(
    _tc_norm_body,
    grid=(GSTEPS,),
    in_specs=[
        pl.BlockSpec((R, D), lambda i: (i, 0)),
        pl.BlockSpec((R, D), lambda i: (i, 0)),
        pl.BlockSpec((8, D), lambda i: (0, 0)),
        pl.BlockSpec((D,), lambda i: (0,)),
        pl.BlockSpec((D,), lambda i: (0,)),
    ],
    out_specs=pl.BlockSpec((R, D), lambda i: (i, 0)),
    out_shape=jax.ShapeDtypeStruct((N, D), jnp.float32),
)


def kernel(x, edge_index, W_l, b_l, W_r, gamma, beta):
    src = edge_index[0].astype(jnp.int32).reshape(NS, CH, C)
    dst = edge_index[1].astype(jnp.int32).reshape(NS, CH, C)
    xh = x.reshape(N, NC, DH).transpose(1, 0, 2)
    aggp, degp = _sc_aggregate(xh, src, dst)
    h, stats = _tc_h(aggp, degp, x, W_l, b_l, W_r)
    return _tc_norm(h, x, stats, gamma, beta)


# speedup vs baseline: 12.1392x; 1.3467x over previous
import functools

import jax
import jax.numpy as jnp
from jax import lax
from jax.experimental import pallas as pl
from jax.experimental.pallas import tpu as pltpu
from jax.experimental.pallas import tpu_sc as plsc

N = 10000
E = 320000
D = 128

NC = 2
NS = 16
DH = D // NC
C = 80
CH = E // (NS * C)
NPAD = 10240
ROWS_PER_TILE = NPAD // NS


def _sc_aggregate_body(xh_hbm, src_hbm, dst_hbm, agg_out, deg_out,
                       src_v, dst_v, rows0_v, rows1_v, rows2_v, rows3_v,
                       ones_v, zeros_v, agg_sh, deg_sh,
                       sem0, sem1, sem2, sem3):
    cid = lax.axis_index("c")
    sid = lax.axis_index("s")

    pltpu.sync_copy(src_hbm.at[sid], src_v)
    pltpu.sync_copy(dst_hbm.at[sid], dst_v)

    zeros16 = jnp.zeros((16,), jnp.float32)
    ones16 = jnp.ones((16,), jnp.float32)

    def fill_row(r, _):
        def fill_col(k, _):
            rows0_v[r, pl.ds(k * 16, 16)] = zeros16
            return 0
        lax.fori_loop(0, DH // 16, fill_col, 0)
        ones_v[r, pl.ds(0, 16)] = ones16
        zeros_v[r, pl.ds(0, 16)] = zeros16
        return 0
    lax.fori_loop(0, C, fill_row, 0)

    for j in range(ROWS_PER_TILE // C):
        pltpu.sync_copy(rows0_v, agg_sh.at[pl.ds(sid * ROWS_PER_TILE + j * C, C)])
        pltpu.sync_copy(zeros_v, deg_sh.at[pl.ds(sid * ROWS_PER_TILE + j * C, C)])
    plsc.subcore_barrier()

    bufs = (rows0_v, rows1_v, rows2_v, rows3_v)
    sems = (sem0, sem1, sem2, sem3)

    def _gather(j, b):
        pltpu.async_copy(xh_hbm.at[cid].at[src_v.at[j]], bufs[b], sems[b])

    def _wait(j, b):
        pltpu.make_async_copy(xh_hbm.at[cid].at[src_v.at[j]], bufs[b],
                              sems[b]).wait()

    def _process(j, b):
        _wait(j, b)
        pltpu.sync_copy(bufs[b], agg_sh.at[dst_v.at[j]], add=True)

        @pl.when(cid == (j % 2))
        def _deg():
            pltpu.sync_copy(ones_v, deg_sh.at[dst_v.at[j]], add=True)

    for b in range(3):
        _gather(b, b)

    def quad(q, _):
        for b in range(4):
            j = 4 * q + b
            jn = j + 3

            @pl.when(jn < CH)
            def _fire():
                _gather(jn, (b + 3) % 4)
            _wait(j, b)
            pltpu.sync_copy(bufs[b], agg_sh.at[dst_v.at[j]], add=True)

            @pl.when(cid == (b % 2))
            def _deg():
                pltpu.sync_copy(ones_v, deg_sh.at[dst_v.at[j]], add=True)
        return 0
    lax.fori_loop(0, CH // 4, quad, 0)
    for j, b in ((CH - 2, 0), (CH - 1, 1)):
        _process(j, b)

    plsc.subcore_barrier()

    pltpu.sync_copy(agg_sh.at[pl.ds(sid * ROWS_PER_TILE, ROWS_PER_TILE)],
                    agg_out.at[cid, pl.ds(sid * ROWS_PER_TILE, ROWS_PER_TILE)])
    pltpu.sync_copy(deg_sh.at[pl.ds(sid * ROWS_PER_TILE, ROWS_PER_TILE)],
                    deg_out.at[cid, pl.ds(sid * ROWS_PER_TILE, ROWS_PER_TILE)])


_sc_aggregate = functools.partial(
    pl.kernel,
    out_type=(jax.ShapeDtypeStruct((NC, NPAD, DH), jnp.float32),
              jax.ShapeDtypeStruct((NC, NPAD, 16), jnp.float32)),
    mesh=plsc.VectorSubcoreMesh(core_axis_name="c", subcore_axis_name="s",
                                num_cores=NC, num_subcores=NS),
    scratch_types=[
        pltpu.VMEM((CH, C), jnp.int32),
        pltpu.VMEM((CH, C), jnp.int32),
        pltpu.VMEM((C, DH), jnp.float32),
        pltpu.VMEM((C, DH), jnp.float32),
        pltpu.VMEM((C, DH), jnp.float32),
        pltpu.VMEM((C, DH), jnp.float32),
        pltpu.VMEM((C, 16), jnp.float32),
        pltpu.VMEM((C, 16), jnp.float32),
        pltpu.VMEM_SHARED((NPAD, DH), jnp.float32),
        pltpu.VMEM_SHARED((NPAD, 16), jnp.float32),
        pltpu.SemaphoreType.DMA,
        pltpu.SemaphoreType.DMA,
        pltpu.SemaphoreType.DMA,
        pltpu.SemaphoreType.DMA,
    ],
    compiler_params=pltpu.CompilerParams(use_tc_tiling_on_sc=False),
)(_sc_aggregate_body)


R = 2000
GSTEPS = N // R


def _tc_h_body(aggp_ref, degp_ref, x_ref, wl_ref, bl_ref, wr_ref,
               h_ref, stats_ref):
    i = pl.program_id(0)
    agg = jnp.concatenate([aggp_ref[0], aggp_ref[1]], axis=1)
    deg = (degp_ref[0] + degp_ref[1])[:, 0:1]
    x = x_ref[...]
    mean_agg = agg * (1.0 / jnp.maximum(deg, 1.0))
    dn = (((1,), (1,)), ((), ()))
    h = (lax.dot_general(mean_agg, wl_ref[...], dn,
                         precision=lax.Precision.HIGHEST,
                         preferred_element_type=jnp.float32)
         + bl_ref[...][None, :]
         + lax.dot_general(x, wr_ref[...], dn,
                           precision=lax.Precision.HIGHEST,
                           preferred_element_type=jnp.float32))
    h_ref[...] = h
    s1 = jnp.sum(h, axis=0, keepdims=True)
    s2 = jnp.sum(h * h, axis=0, keepdims=True)
    part = jnp.concatenate(
        [s1, s2, jnp.zeros((6, D), jnp.float32)], axis=0)

    @pl.when(i == 0)
    def _init():
        stats_ref[...] = part

    @pl.when(i > 0)
    def _acc():
        stats_ref[...] += part


_tc_h = pl.pallas_call(
    _tc_h_body,
    grid=(GSTEPS,),
    in_specs=[
        pl.BlockSpec((NC, R, DH), lambda i: (0, i, 0)),
        pl.BlockSpec((NC, R, 16), lambda i: (0, i, 0)),
        pl.BlockSpec((R, D), lambda i: (i, 0)),
        pl.BlockSpec((D, D), lambda i: (0, 0)),
        pl.BlockSpec((D,), lambda i: (0,)),
        pl.BlockSpec((D, D), lambda i: (0, 0)),
    ],
    out_specs=(
        pl.BlockSpec((R, D), lambda i: (i, 0)),
        pl.BlockSpec((8, D), lambda i: (0, 0)),
    ),
    out_shape=(jax.ShapeDtypeStruct((N, D), jnp.float32),
               jax.ShapeDtypeStruct((8, D), jnp.float32)),
)


def _tc_norm_body(h_ref, x_ref, stats_ref, g_ref, b_ref, o_ref):
    h = h_ref[...]
    mu = stats_ref[0:1, :] * (1.0 / N)
    var = stats_ref[1:2, :] * (1.0 / N) - mu * mu
    hn = (h - mu) * lax.rsqrt(var + 1e-5) * g_ref[...][None, :] + b_ref[...][None, :]
    o_ref[...] = jnp.maximum(hn, 0.0) + x_ref[...]


_tc_norm = pl.pallas_call(
    _tc_norm_body,
    grid=(GSTEPS,),
    in_specs=[
        pl.BlockSpec((R, D), lambda i: (i, 0)),
        pl.BlockSpec((R, D), lambda i: (i, 0)),
        pl.BlockSpec((8, D), lambda i: (0, 0)),
        pl.BlockSpec((D,), lambda i: (0,)),
        pl.BlockSpec((D,), lambda i: (0,)),
    ],
    out_specs=pl.BlockSpec((R, D), lambda i: (i, 0)),
    out_shape=jax.ShapeDtypeStruct((N, D), jnp.float32),
)


def kernel(x, edge_index, W_l, b_l, W_r, gamma, beta):
    src = edge_index[0].astype(jnp.int32).reshape(NS, CH, C)
    dst = edge_index[1].astype(jnp.int32).reshape(NS, CH, C)
    xh = x.reshape(N, NC, DH).transpose(1, 0, 2)
    aggp, degp = _sc_aggregate(xh, src, dst)
    h, stats = _tc_h(aggp, degp, x, W_l, b_l, W_r)
    return _tc_norm(h, x, stats, gamma, beta)


# speedup vs baseline: 12.4885x; 1.0288x over previous
import functools

import jax
import jax.numpy as jnp
from jax import lax
from jax.experimental import pallas as pl
from jax.experimental.pallas import tpu as pltpu
from jax.experimental.pallas import tpu_sc as plsc

N = 10000
E = 320000
D = 128

NC = 2
NS = 16
DH = D // NC
C = 80
CH = E // (NS * C)
NPAD = 10240
ROWS_PER_TILE = NPAD // NS


def _sc_aggregate_body(xh_hbm, src_hbm, dst_hbm, agg_out, deg_out,
                       src_v, dst_v, rows0_v, rows1_v, rows2_v, rows3_v,
                       ones_v, zeros_v, agg_sh, deg_sh,
                       sem0, sem1, sem2, sem3):
    cid = lax.axis_index("c")
    sid = lax.axis_index("s")

    pltpu.sync_copy(src_hbm.at[sid], src_v)
    pltpu.sync_copy(dst_hbm.at[sid], dst_v)

    zeros16 = jnp.zeros((16,), jnp.float32)
    ones16 = jnp.ones((16,), jnp.float32)

    def fill_row(r, _):
        def fill_col(k, _):
            rows0_v[r, pl.ds(k * 16, 16)] = zeros16
            return 0
        lax.fori_loop(0, DH // 16, fill_col, 0)
        ones_v[r, pl.ds(0, 16)] = ones16
        zeros_v[r, pl.ds(0, 16)] = zeros16
        return 0
    lax.fori_loop(0, C, fill_row, 0)

    for j in range(ROWS_PER_TILE // C):
        pltpu.sync_copy(rows0_v, agg_sh.at[pl.ds(sid * ROWS_PER_TILE + j * C, C)])
        pltpu.sync_copy(zeros_v, deg_sh.at[pl.ds(sid * ROWS_PER_TILE + j * C, C)])
    plsc.subcore_barrier()

    bufs = (rows0_v, rows1_v, rows2_v, rows3_v)
    sems = (sem0, sem1, sem2, sem3)

    def _gather(j, b):
        pltpu.async_copy(xh_hbm.at[cid].at[src_v.at[j]], bufs[b], sems[b])

    def _wait(j, b):
        pltpu.make_async_copy(xh_hbm.at[cid].at[src_v.at[j]], bufs[b],
                              sems[b]).wait()

    def _process(j, b):
        _wait(j, b)
        pltpu.sync_copy(bufs[b], agg_sh.at[dst_v.at[j]], add=True)

        @pl.when(cid == (j % 2))
        def _deg():
            pltpu.sync_copy(ones_v, deg_sh.at[dst_v.at[j]], add=True)

    for b in range(3):
        _gather(b, b)

    def quad(q, _):
        for b in range(4):
            j = 4 * q + b
            jn = j + 3

            @pl.when(jn < CH)
            def _fire():
                _gather(jn, (b + 3) % 4)
            _wait(j, b)
            pltpu.sync_copy(bufs[b], agg_sh.at[dst_v.at[j]], add=True)

            @pl.when(cid == (b % 2))
            def _deg():
                pltpu.sync_copy(ones_v, deg_sh.at[dst_v.at[j]], add=True)
        return 0
    lax.fori_loop(0, CH // 4, quad, 0)
    for j, b in ((CH - 2, 0), (CH - 1, 1)):
        _process(j, b)

    plsc.subcore_barrier()

    pltpu.sync_copy(agg_sh.at[pl.ds(sid * ROWS_PER_TILE, ROWS_PER_TILE)],
                    agg_out.at[cid, pl.ds(sid * ROWS_PER_TILE, ROWS_PER_TILE)])
    pltpu.sync_copy(deg_sh.at[pl.ds(sid * ROWS_PER_TILE, ROWS_PER_TILE)],
                    deg_out.at[cid, pl.ds(sid * ROWS_PER_TILE, ROWS_PER_TILE)])


_sc_aggregate = functools.partial(
    pl.kernel,
    out_type=(jax.ShapeDtypeStruct((NC, NPAD, DH), jnp.float32),
              jax.ShapeDtypeStruct((NC, NPAD, 16), jnp.float32)),
    mesh=plsc.VectorSubcoreMesh(core_axis_name="c", subcore_axis_name="s",
                                num_cores=NC, num_subcores=NS),
    scratch_types=[
        pltpu.VMEM((CH, C), jnp.int32),
        pltpu.VMEM((CH, C), jnp.int32),
        pltpu.VMEM((C, DH), jnp.float32),
        pltpu.VMEM((C, DH), jnp.float32),
        pltpu.VMEM((C, DH), jnp.float32),
        pltpu.VMEM((C, DH), jnp.float32),
        pltpu.VMEM((C, 16), jnp.float32),
        pltpu.VMEM((C, 16), jnp.float32),
        pltpu.VMEM_SHARED((NPAD, DH), jnp.float32),
        pltpu.VMEM_SHARED((NPAD, 16), jnp.float32),
        pltpu.SemaphoreType.DMA,
        pltpu.SemaphoreType.DMA,
        pltpu.SemaphoreType.DMA,
        pltpu.SemaphoreType.DMA,
    ],
    compiler_params=pltpu.CompilerParams(use_tc_tiling_on_sc=False),
)(_sc_aggregate_body)


R = 2000
GSTEPS = N // R


def _tc_hr_body(x_ref, wr_ref, bl_ref, o_ref):
    dn = (((1,), (1,)), ((), ()))
    o_ref[...] = (lax.dot_general(x_ref[...], wr_ref[...], dn,
                                  precision=lax.Precision.HIGHEST,
                                  preferred_element_type=jnp.float32)
                  + bl_ref[...][None, :])


_tc_hr = pl.pallas_call(
    _tc_hr_body,
    grid=(GSTEPS,),
    in_specs=[
        pl.BlockSpec((R, D), lambda i: (i, 0)),
        pl.BlockSpec((D, D), lambda i: (0, 0)),
        pl.BlockSpec((D,), lambda i: (0,)),
    ],
    out_specs=pl.BlockSpec((R, D), lambda i: (i, 0)),
    out_shape=jax.ShapeDtypeStruct((N, D), jnp.float32),
)


def _tc_finish_body(aggp_ref, degp_ref, hr_ref, x_ref, wl_ref,
                    g_ref, b_ref, o_ref, h_scr, st_scr):
    i = pl.program_id(0)
    blk = jnp.where(i < GSTEPS, i, i - GSTEPS)
    row0 = pl.multiple_of(blk * R, R)

    @pl.when(i < GSTEPS)
    def _phase_h():
        agg = jnp.concatenate([aggp_ref[0], aggp_ref[1]], axis=1)
        deg = (degp_ref[0] + degp_ref[1])[:, 0:1]
        mean_agg = agg * (1.0 / jnp.maximum(deg, 1.0))
        dn = (((1,), (1,)), ((), ()))
        h = (lax.dot_general(mean_agg, wl_ref[...], dn,
                             precision=lax.Precision.HIGHEST,
                             preferred_element_type=jnp.float32)
             + hr_ref[...])
        h_scr[pl.ds(row0, R), :] = h
        s1 = jnp.sum(h, axis=0, keepdims=True)
        s2 = jnp.sum(h * h, axis=0, keepdims=True)
        part = jnp.concatenate(
            [s1, s2, jnp.zeros((6, D), jnp.float32)], axis=0)

        @pl.when(i == 0)
        def _init():
            st_scr[...] = part

        @pl.when(i > 0)
        def _acc():
            st_scr[...] += part
        o_ref[...] = h

    @pl.when(i >= GSTEPS)
    def _phase_norm():
        h = h_scr[pl.ds(row0, R), :]
        mu = st_scr[0:1, :] * (1.0 / N)
        var = st_scr[1:2, :] * (1.0 / N) - mu * mu
        hn = ((h - mu) * lax.rsqrt(var + 1e-5) * g_ref[...][None, :]
              + b_ref[...][None, :])
        o_ref[...] = jnp.maximum(hn, 0.0) + x_ref[...]


_tc_finish = pl.pallas_call(
    _tc_finish_body,
    grid=(2 * GSTEPS,),
    in_specs=[
        pl.BlockSpec((NC, R, DH), lambda i: (0, jnp.minimum(i, GSTEPS - 1), 0)),
        pl.BlockSpec((NC, R, 16), lambda i: (0, jnp.minimum(i, GSTEPS - 1), 0)),
        pl.BlockSpec((R, D), lambda i: (jnp.minimum(i, GSTEPS - 1), 0)),
        pl.BlockSpec((R, D), lambda i: (jnp.maximum(i, GSTEPS) - GSTEPS, 0)),
        pl.BlockSpec((D, D), lambda i: (0, 0)),
        pl.BlockSpec((D,), lambda i: (0,)),
        pl.BlockSpec((D,), lambda i: (0,)),
    ],
    out_specs=pl.BlockSpec((R, D), lambda i: (jnp.maximum(i, GSTEPS) - GSTEPS, 0)),
    out_shape=jax.ShapeDtypeStruct((N, D), jnp.float32),
    scratch_shapes=[
        pltpu.VMEM((N, D), jnp.float32),
        pltpu.VMEM((8, D), jnp.float32),
    ],
)


def kernel(x, edge_index, W_l, b_l, W_r, gamma, beta):
    src = edge_index[0].astype(jnp.int32).reshape(NS, CH, C)
    dst = edge_index[1].astype(jnp.int32).reshape(NS, CH, C)
    xh = x.reshape(N, NC, DH).transpose(1, 0, 2)
    aggp, degp = _sc_aggregate(xh, src, dst)
    hr = _tc_hr(x, W_r, b_l)
    return _tc_finish(aggp, degp, hr, x, W_l, gamma, beta)
